# baseline (speedup 1.0000x reference)
import functools

import jax
import jax.numpy as jnp
from jax import lax
from jax.experimental import pallas as pl
from jax.experimental.pallas import tpu as pltpu
from jax.experimental.pallas import tpu_sc as plsc

BATCH = 16384
EMBED_DIM = 64
LANES = 16
NUM_CORES = 2
NUM_SUBCORES = 16
NUM_WORKERS = NUM_CORES * NUM_SUBCORES
BPW = BATCH // NUM_WORKERS
RPB = 16
NB = BPW // RPB
SUBROWS = 8


def _fire(emb_hbm, idx_ref, blocks, sem, k):
    nvec = idx_ref[pl.ds(k * RPB, RPB)]
    base = lax.shift_right_logical(nvec, 3) * SUBROWS
    for t in range(RPB):
        pltpu.async_copy(
            emb_hbm.at[pl.ds(pl.multiple_of(base[t], SUBROWS), SUBROWS), :],
            blocks.at[t], sem)


def _drain(emb_hbm, blocks, sem):
    for t in range(RPB):
        pltpu.make_async_copy(
            emb_hbm.at[pl.ds(0, SUBROWS), :], blocks.at[t], sem).wait()


def _dot_body(i_hbm, j_hbm, emb_hbm, out_hbm,
              idx_i, idx_j, bi0, bi1, bj0, bj1, out_v,
              si0, si1, sj0, sj1):
    c = lax.axis_index("c")
    s = lax.axis_index("s")
    wid = s * NUM_CORES + c
    base_row = wid * BPW

    pltpu.sync_copy(i_hbm.at[wid], idx_i)
    pltpu.sync_copy(j_hbm.at[wid], idx_j)

    bufs_i = (bi0, bi1)
    bufs_j = (bj0, bj1)
    sems_i = (si0, si1)
    sems_j = (sj0, sj1)

    _fire(emb_hbm, idx_i, bufs_i[0], sems_i[0], 0)
    _fire(emb_hbm, idx_j, bufs_j[0], sems_j[0], 0)
    _fire(emb_hbm, idx_i, bufs_i[1], sems_i[1], 1)
    _fire(emb_hbm, idx_j, bufs_j[1], sems_j[1], 1)

    lane = lax.broadcasted_iota(jnp.int32, (LANES,), 0)

    def step(k2, _):
        for b in range(2):
            k = 2 * k2 + b
            bi, bj = bufs_i[b], bufs_j[b]
            _drain(emb_hbm, bi, sems_i[b])
            _drain(emb_hbm, bj, sems_j[b])
            nv_i = idx_i[pl.ds(k * RPB, RPB)] & 7
            nv_j = idx_j[pl.ds(k * RPB, RPB)] & 7
            out_vec = jnp.zeros((LANES,), jnp.float32)
            for t in range(RPB):
                si_t = nv_i[t]
                sj_t = nv_j[t]
                acc = jnp.zeros((LANES,), jnp.float32)
                for d in range(EMBED_DIM // LANES):
                    vi = bi[t, si_t, pl.ds(d * LANES, LANES)]
                    vj = bj[t, sj_t, pl.ds(d * LANES, LANES)]
                    acc = acc + vi * vj
                dot = jnp.sum(acc)
                out_vec = jnp.where(lane == t, dot, out_vec)
            out_v[pl.ds(k * RPB, RPB)] = out_vec

            @pl.when(k + 2 < NB)
            def _():
                _fire(emb_hbm, idx_i, bi, sems_i[b], k + 2)
                _fire(emb_hbm, idx_j, bj, sems_j[b], k + 2)
        return 0

    lax.fori_loop(0, NB // 2, step, 0)

    pltpu.sync_copy(out_v, out_hbm.at[pl.ds(base_row, BPW)])


@jax.jit
def _sc_dot(i, j, node_emb):
    mesh = plsc.VectorSubcoreMesh(core_axis_name="c", subcore_axis_name="s")
    kfn = pl.kernel(
        _dot_body,
        mesh=mesh,
        compiler_params=pltpu.CompilerParams(
            needs_layout_passes=False, use_tc_tiling_on_sc=True),
        out_type=jax.ShapeDtypeStruct((BATCH,), jnp.float32),
        scratch_types=[
            pltpu.VMEM((BPW,), jnp.int32),
            pltpu.VMEM((BPW,), jnp.int32),
            pltpu.VMEM((RPB, SUBROWS, EMBED_DIM), jnp.float32),
            pltpu.VMEM((RPB, SUBROWS, EMBED_DIM), jnp.float32),
            pltpu.VMEM((RPB, SUBROWS, EMBED_DIM), jnp.float32),
            pltpu.VMEM((RPB, SUBROWS, EMBED_DIM), jnp.float32),
            pltpu.VMEM((BPW,), jnp.float32),
            pltpu.SemaphoreType.DMA,
            pltpu.SemaphoreType.DMA,
            pltpu.SemaphoreType.DMA,
            pltpu.SemaphoreType.DMA,
        ],
    )
    return kfn(i.reshape(NUM_WORKERS, BPW), j.reshape(NUM_WORKERS, BPW),
               node_emb)


def kernel(i, j, node_emb):
    return _sc_dot(i.astype(jnp.int32), j.astype(jnp.int32), node_emb)


# speedup vs baseline: 1.1880x; 1.1880x over previous
import functools

import jax
import jax.numpy as jnp
from jax import lax
from jax.experimental import pallas as pl
from jax.experimental.pallas import tpu as pltpu
from jax.experimental.pallas import tpu_sc as plsc

BATCH = 16384
EMBED_DIM = 64
LANES = 16
NUM_CORES = 2
NUM_SUBCORES = 16
NUM_WORKERS = NUM_CORES * NUM_SUBCORES
BPW = BATCH // NUM_WORKERS
RPB = 16
NB = BPW // RPB
SUBROWS = 8


def _fire(emb_hbm, idx_ref, blocks, sem, k):
    nvec = idx_ref[pl.ds(k * RPB, RPB)]
    base = lax.shift_right_logical(nvec, 3) * SUBROWS
    for t in range(RPB):
        pltpu.async_copy(
            emb_hbm.at[pl.ds(pl.multiple_of(base[t], SUBROWS), SUBROWS), :],
            blocks.at[t], sem)


def _drain(emb_hbm, blocks, sem):
    for t in range(RPB):
        pltpu.make_async_copy(
            emb_hbm.at[pl.ds(0, SUBROWS), :], blocks.at[t], sem).wait()


def _dot_body(i_hbm, j_hbm, emb_hbm, out_hbm,
              idx_i, idx_j, bi0, bi1, bj0, bj1, out_v,
              si0, si1, sj0, sj1):
    c = lax.axis_index("c")
    s = lax.axis_index("s")
    wid = s * NUM_CORES + c
    base_row = wid * BPW

    pltpu.sync_copy(i_hbm.at[wid], idx_i)
    pltpu.sync_copy(j_hbm.at[wid], idx_j)

    bufs_i = (bi0, bi1)
    bufs_j = (bj0, bj1)
    sems_i = (si0, si1)
    sems_j = (sj0, sj1)

    _fire(emb_hbm, idx_i, bufs_i[0], sems_i[0], 0)
    _fire(emb_hbm, idx_j, bufs_j[0], sems_j[0], 0)
    _fire(emb_hbm, idx_i, bufs_i[1], sems_i[1], 1)
    _fire(emb_hbm, idx_j, bufs_j[1], sems_j[1], 1)

    lane = lax.broadcasted_iota(jnp.int32, (LANES,), 0)

    def step(k2, _):
        for b in range(2):
            k = 2 * k2 + b
            bi, bj = bufs_i[b], bufs_j[b]
            _drain(emb_hbm, bi, sems_i[b])
            _drain(emb_hbm, bj, sems_j[b])
            nv_i = idx_i[pl.ds(k * RPB, RPB)] & 7
            nv_j = idx_j[pl.ds(k * RPB, RPB)] & 7
            out_vec = jnp.zeros((LANES,), jnp.float32)
            for t in range(RPB):
                si_t = nv_i[t]
                sj_t = nv_j[t]
                acc = jnp.zeros((LANES,), jnp.float32)
                for d in range(EMBED_DIM // LANES):
                    vi = bi[t, si_t, pl.ds(d * LANES, LANES)]
                    vj = bj[t, sj_t, pl.ds(d * LANES, LANES)]
                    acc = acc + vi * vj
                dot = jnp.sum(acc)
                out_vec = jnp.where(lane == t, dot, out_vec)
            out_v[pl.ds(k * RPB, RPB)] = out_vec

            @pl.when(k + 2 < NB)
            def _():
                _fire(emb_hbm, idx_i, bi, sems_i[b], k + 2)
                _fire(emb_hbm, idx_j, bj, sems_j[b], k + 2)
        return 0

    lax.fori_loop(0, NB // 2, step, 0)

    pltpu.sync_copy(out_v, out_hbm.at[pl.ds(base_row, BPW)])


TR_BK = 8192


def _tr_body(x_ref, out_ref):
    d0 = lax.broadcasted_iota(jnp.int32, (EMBED_DIM, EMBED_DIM), 0)
    d1 = lax.broadcasted_iota(jnp.int32, (EMBED_DIM, EMBED_DIM), 1)
    eye = (d0 == d1).astype(jnp.float32)
    out_ref[...] = lax.dot_general(
        x_ref[...], eye, (((0,), (0,)), ((), ())),
        preferred_element_type=jnp.float32)


def _transpose_table(emb_t):
    num_nodes = emb_t.shape[1]
    grid = (num_nodes + TR_BK - 1) // TR_BK
    return pl.pallas_call(
        _tr_body,
        grid=(grid,),
        in_specs=[pl.BlockSpec((EMBED_DIM, TR_BK), lambda g: (0, g))],
        out_specs=pl.BlockSpec((TR_BK, EMBED_DIM), lambda g: (g, 0)),
        out_shape=jax.ShapeDtypeStruct((num_nodes, EMBED_DIM), jnp.float32),
    )(emb_t)


@jax.jit
def _sc_dot(i, j, node_emb):
    mesh = plsc.VectorSubcoreMesh(core_axis_name="c", subcore_axis_name="s")
    kfn = pl.kernel(
        _dot_body,
        mesh=mesh,
        compiler_params=pltpu.CompilerParams(
            needs_layout_passes=False, use_tc_tiling_on_sc=True),
        out_type=jax.ShapeDtypeStruct((BATCH,), jnp.float32),
        scratch_types=[
            pltpu.VMEM((BPW,), jnp.int32),
            pltpu.VMEM((BPW,), jnp.int32),
            pltpu.VMEM((RPB, SUBROWS, EMBED_DIM), jnp.float32),
            pltpu.VMEM((RPB, SUBROWS, EMBED_DIM), jnp.float32),
            pltpu.VMEM((RPB, SUBROWS, EMBED_DIM), jnp.float32),
            pltpu.VMEM((RPB, SUBROWS, EMBED_DIM), jnp.float32),
            pltpu.VMEM((BPW,), jnp.float32),
            pltpu.SemaphoreType.DMA,
            pltpu.SemaphoreType.DMA,
            pltpu.SemaphoreType.DMA,
            pltpu.SemaphoreType.DMA,
        ],
    )
    emb_row = _transpose_table(node_emb.T)
    return kfn(i.reshape(NUM_WORKERS, BPW), j.reshape(NUM_WORKERS, BPW),
               emb_row)


def kernel(i, j, node_emb):
    return _sc_dot(i.astype(jnp.int32), j.astype(jnp.int32), node_emb)


# speedup vs baseline: 1.4294x; 1.2033x over previous
import functools

import jax
import jax.numpy as jnp
from jax import lax
from jax.experimental import pallas as pl
from jax.experimental.pallas import tpu as pltpu
from jax.experimental.pallas import tpu_sc as plsc

BATCH = 16384
EMBED_DIM = 64
PACK_DIM = 128
LANES = 16
NUM_CORES = 2
NUM_SUBCORES = 16
NUM_WORKERS = NUM_CORES * NUM_SUBCORES
BPW = BATCH // NUM_WORKERS
CHUNK = 128
NCHUNKS = BPW // CHUNK
GROUPS = CHUNK // LANES

TR_BK = 8192
TR_HALF = TR_BK // 2
NUM_NODES = 1000000
TR_GRID = (NUM_NODES + TR_BK - 1) // TR_BK
PACK_ROWS = TR_GRID * TR_HALF


def _tr_body(x_ref, out_ref):
    d0 = lax.broadcasted_iota(jnp.int32, (EMBED_DIM, EMBED_DIM), 0)
    d1 = lax.broadcasted_iota(jnp.int32, (EMBED_DIM, EMBED_DIM), 1)
    eye = (d0 == d1).astype(jnp.float32)
    dn = (((0,), (0,)), ((), ()))
    left = lax.dot_general(x_ref[:, :TR_HALF], eye, dn,
                           preferred_element_type=jnp.float32)
    right = lax.dot_general(x_ref[:, TR_HALF:], eye, dn,
                            preferred_element_type=jnp.float32)
    out_ref[...] = jnp.concatenate([left, right], axis=1)


def _pack_table(emb_t):
    return pl.pallas_call(
        _tr_body,
        grid=(TR_GRID,),
        in_specs=[pl.BlockSpec((EMBED_DIM, TR_BK), lambda g: (0, g))],
        out_specs=pl.BlockSpec((TR_HALF, PACK_DIM), lambda g: (g, 0)),
        out_shape=jax.ShapeDtypeStruct((PACK_ROWS, PACK_DIM), jnp.float32),
    )(emb_t)


def _packed_row(n):
    return lax.shift_left(lax.shift_right_logical(n, 13), 12) + (n & 4095)


def _dot_body(i_hbm, j_hbm, emb_hbm, out_hbm,
              idx_i, idx_j, sup_i, sup_j,
              bi0, bi1, bj0, bj1, out_v,
              si0, si1, sj0, sj1):
    c = lax.axis_index("c")
    s = lax.axis_index("s")
    wid = s * NUM_CORES + c

    pltpu.sync_copy(i_hbm.at[wid], idx_i)
    pltpu.sync_copy(j_hbm.at[wid], idx_j)

    for k in range(NCHUNKS):
        for t in range(CHUNK // LANES):
            sl = pl.ds(t * LANES, LANES)
            sup_i[k, sl] = _packed_row(idx_i[k, sl])
            sup_j[k, sl] = _packed_row(idx_j[k, sl])

    bufs_i = (bi0, bi1)
    bufs_j = (bj0, bj1)
    sems_i = (si0, si1)
    sems_j = (sj0, sj1)

    def fire(k):
        b = k % 2
        return (pltpu.async_copy(emb_hbm.at[sup_i.at[k]], bufs_i[b], sems_i[b]),
                pltpu.async_copy(emb_hbm.at[sup_j.at[k]], bufs_j[b], sems_j[b]))

    lane = lax.broadcasted_iota(jnp.int32, (LANES,), 0)
    inflight = fire(0)

    for k in range(NCHUNKS):
        b = k % 2
        for cp in inflight:
            cp.wait()
        if k + 1 < NCHUNKS:
            inflight = fire(k + 1)
        bi, bj = bufs_i[b], bufs_j[b]

        def group(g, _):
            sl = pl.ds(g * LANES, LANES)
            hi = (lax.shift_right_logical(idx_i[k, sl], 12) & 1) * EMBED_DIM
            hj = (lax.shift_right_logical(idx_j[k, sl], 12) & 1) * EMBED_DIM
            out_vec = jnp.zeros((LANES,), jnp.float32)
            for t in range(LANES):
                r = g * LANES + t
                hb_i = hi[t]
                hb_j = hj[t]
                acc = jnp.zeros((LANES,), jnp.float32)
                for d in range(EMBED_DIM // LANES):
                    vi = bi[r, pl.ds(hb_i + d * LANES, LANES)]
                    vj = bj[r, pl.ds(hb_j + d * LANES, LANES)]
                    acc = acc + vi * vj
                dot = jnp.sum(acc)
                out_vec = jnp.where(lane == t, dot, out_vec)
            out_v[pl.ds(k * CHUNK + g * LANES, LANES)] = out_vec
            return 0

        lax.fori_loop(0, GROUPS, group, 0)

    pltpu.sync_copy(out_v, out_hbm.at[pl.ds(wid * BPW, BPW)])


@jax.jit
def _sc_dot(i, j, node_emb):
    mesh = plsc.VectorSubcoreMesh(core_axis_name="c", subcore_axis_name="s")
    kfn = pl.kernel(
        _dot_body,
        mesh=mesh,
        compiler_params=pltpu.CompilerParams(
            needs_layout_passes=False, use_tc_tiling_on_sc=True),
        out_type=jax.ShapeDtypeStruct((BATCH,), jnp.float32),
        scratch_types=[
            pltpu.VMEM((NCHUNKS, CHUNK), jnp.int32),
            pltpu.VMEM((NCHUNKS, CHUNK), jnp.int32),
            pltpu.VMEM((NCHUNKS, CHUNK), jnp.int32),
            pltpu.VMEM((NCHUNKS, CHUNK), jnp.int32),
            pltpu.VMEM((CHUNK, PACK_DIM), jnp.float32),
            pltpu.VMEM((CHUNK, PACK_DIM), jnp.float32),
            pltpu.VMEM((CHUNK, PACK_DIM), jnp.float32),
            pltpu.VMEM((CHUNK, PACK_DIM), jnp.float32),
            pltpu.VMEM((BPW,), jnp.float32),
            pltpu.SemaphoreType.DMA,
            pltpu.SemaphoreType.DMA,
            pltpu.SemaphoreType.DMA,
            pltpu.SemaphoreType.DMA,
        ],
    )
    packed = _pack_table(node_emb.T)
    return kfn(i.reshape(NUM_WORKERS, NCHUNKS, CHUNK),
               j.reshape(NUM_WORKERS, NCHUNKS, CHUNK),
               packed)


def kernel(i, j, node_emb):
    return _sc_dot(i.astype(jnp.int32), j.astype(jnp.int32), node_emb)


# speedup vs baseline: 1.4344x; 1.0035x over previous
import functools

import jax
import jax.numpy as jnp
from jax import lax
from jax.experimental import pallas as pl
from jax.experimental.pallas import tpu as pltpu
from jax.experimental.pallas import tpu_sc as plsc

BATCH = 16384
EMBED_DIM = 64
PACK_DIM = 128
LANES = 16
NUM_CORES = 2
NUM_SUBCORES = 16
NUM_WORKERS = NUM_CORES * NUM_SUBCORES
BPW = BATCH // NUM_WORKERS
CHUNK = 128
NCHUNKS = BPW // CHUNK
GROUPS = CHUNK // LANES

TR_BK = 8192
TR_HALF = TR_BK // 2
NUM_NODES = 1000000
TR_GRID = (NUM_NODES + TR_BK - 1) // TR_BK
PACK_ROWS = TR_GRID * TR_HALF


def _tr_body(x_ref, out_ref):
    xt = lax.transpose(x_ref[...], (1, 0))
    out_ref[...] = jnp.concatenate([xt[:TR_HALF], xt[TR_HALF:]], axis=1)


def _pack_table(emb_t):
    return pl.pallas_call(
        _tr_body,
        grid=(TR_GRID,),
        in_specs=[pl.BlockSpec((EMBED_DIM, TR_BK), lambda g: (0, g))],
        out_specs=pl.BlockSpec((TR_HALF, PACK_DIM), lambda g: (g, 0)),
        out_shape=jax.ShapeDtypeStruct((PACK_ROWS, PACK_DIM), jnp.float32),
    )(emb_t)


def _packed_row(n):
    return lax.shift_left(lax.shift_right_logical(n, 13), 12) + (n & 4095)


def _dot_body(i_hbm, j_hbm, emb_hbm, out_hbm,
              idx_i, idx_j, sup_i, sup_j,
              bi0, bi1, bj0, bj1, out_v,
              si0, si1, sj0, sj1):
    c = lax.axis_index("c")
    s = lax.axis_index("s")
    wid = s * NUM_CORES + c

    pltpu.sync_copy(i_hbm.at[wid], idx_i)
    pltpu.sync_copy(j_hbm.at[wid], idx_j)

    for k in range(NCHUNKS):
        for t in range(CHUNK // LANES):
            sl = pl.ds(t * LANES, LANES)
            sup_i[k, sl] = _packed_row(idx_i[k, sl])
            sup_j[k, sl] = _packed_row(idx_j[k, sl])

    bufs_i = (bi0, bi1)
    bufs_j = (bj0, bj1)
    sems_i = (si0, si1)
    sems_j = (sj0, sj1)

    def fire(k):
        b = k % 2
        return (pltpu.async_copy(emb_hbm.at[sup_i.at[k]], bufs_i[b], sems_i[b]),
                pltpu.async_copy(emb_hbm.at[sup_j.at[k]], bufs_j[b], sems_j[b]))

    lane = lax.broadcasted_iota(jnp.int32, (LANES,), 0)
    inflight = fire(0)

    for k in range(NCHUNKS):
        b = k % 2
        for cp in inflight:
            cp.wait()
        if k + 1 < NCHUNKS:
            inflight = fire(k + 1)
        bi, bj = bufs_i[b], bufs_j[b]

        def group(g, _):
            sl = pl.ds(g * LANES, LANES)
            hi = (lax.shift_right_logical(idx_i[k, sl], 12) & 1) * EMBED_DIM
            hj = (lax.shift_right_logical(idx_j[k, sl], 12) & 1) * EMBED_DIM
            out_vec = jnp.zeros((LANES,), jnp.float32)
            for t in range(LANES):
                r = g * LANES + t
                hb_i = hi[t]
                hb_j = hj[t]
                acc = jnp.zeros((LANES,), jnp.float32)
                for d in range(EMBED_DIM // LANES):
                    vi = bi[r, pl.ds(hb_i + d * LANES, LANES)]
                    vj = bj[r, pl.ds(hb_j + d * LANES, LANES)]
                    acc = acc + vi * vj
                dot = jnp.sum(acc)
                out_vec = jnp.where(lane == t, dot, out_vec)
            out_v[pl.ds(k * CHUNK + g * LANES, LANES)] = out_vec
            return 0

        lax.fori_loop(0, GROUPS, group, 0)

    pltpu.sync_copy(out_v, out_hbm.at[pl.ds(wid * BPW, BPW)])


@jax.jit
def _sc_dot(i, j, node_emb):
    mesh = plsc.VectorSubcoreMesh(core_axis_name="c", subcore_axis_name="s")
    kfn = pl.kernel(
        _dot_body,
        mesh=mesh,
        compiler_params=pltpu.CompilerParams(
            needs_layout_passes=False, use_tc_tiling_on_sc=True),
        out_type=jax.ShapeDtypeStruct((BATCH,), jnp.float32),
        scratch_types=[
            pltpu.VMEM((NCHUNKS, CHUNK), jnp.int32),
            pltpu.VMEM((NCHUNKS, CHUNK), jnp.int32),
            pltpu.VMEM((NCHUNKS, CHUNK), jnp.int32),
            pltpu.VMEM((NCHUNKS, CHUNK), jnp.int32),
            pltpu.VMEM((CHUNK, PACK_DIM), jnp.float32),
            pltpu.VMEM((CHUNK, PACK_DIM), jnp.float32),
            pltpu.VMEM((CHUNK, PACK_DIM), jnp.float32),
            pltpu.VMEM((CHUNK, PACK_DIM), jnp.float32),
            pltpu.VMEM((BPW,), jnp.float32),
            pltpu.SemaphoreType.DMA,
            pltpu.SemaphoreType.DMA,
            pltpu.SemaphoreType.DMA,
            pltpu.SemaphoreType.DMA,
        ],
    )
    packed = _pack_table(node_emb.T)
    return kfn(i.reshape(NUM_WORKERS, NCHUNKS, CHUNK),
               j.reshape(NUM_WORKERS, NCHUNKS, CHUNK),
               packed)


def kernel(i, j, node_emb):
    return _sc_dot(i.astype(jnp.int32), j.astype(jnp.int32), node_emb)


# speedup vs baseline: 1.6129x; 1.1244x over previous
import functools

import jax
import jax.numpy as jnp
from jax import lax
from jax.experimental import pallas as pl
from jax.experimental.pallas import tpu as pltpu
from jax.experimental.pallas import tpu_sc as plsc

BATCH = 16384
EMBED_DIM = 64
PACK_DIM = 128
LANES = 16
NUM_CORES = 2
NUM_SUBCORES = 16
NUM_WORKERS = NUM_CORES * NUM_SUBCORES
BPW = BATCH // NUM_WORKERS
CHUNK = 128
NCHUNKS = BPW // CHUNK
GROUPS = CHUNK // LANES

TR_BK = 16384
TR_HALF = TR_BK // 2
NUM_NODES = 1000000
TR_GRID = (NUM_NODES + TR_BK - 1) // TR_BK
PACK_ROWS = TR_GRID * TR_HALF


def _tr_body(x_ref, out_ref):
    xt = lax.transpose(x_ref[...], (1, 0))
    out_ref[...] = jnp.concatenate([xt[:TR_HALF], xt[TR_HALF:]], axis=1)


def _pack_table(emb_t):
    return pl.pallas_call(
        _tr_body,
        grid=(TR_GRID,),
        in_specs=[pl.BlockSpec((EMBED_DIM, TR_BK), lambda g: (0, g))],
        out_specs=pl.BlockSpec((TR_HALF, PACK_DIM), lambda g: (g, 0)),
        out_shape=jax.ShapeDtypeStruct((PACK_ROWS, PACK_DIM), jnp.float32),
    )(emb_t)


TR_BK_LOG2 = TR_BK.bit_length() - 1
TR_HALF_LOG2 = TR_HALF.bit_length() - 1


def _packed_row(n):
    return lax.shift_left(
        lax.shift_right_logical(n, TR_BK_LOG2), TR_HALF_LOG2
    ) + (n & (TR_HALF - 1))


def _dot_body(i_hbm, j_hbm, emb_hbm, out_hbm,
              idx_i, idx_j, sup_i, sup_j,
              bi0, bi1, bj0, bj1, out_v,
              si0, si1, sj0, sj1):
    c = lax.axis_index("c")
    s = lax.axis_index("s")
    wid = s * NUM_CORES + c

    pltpu.sync_copy(i_hbm.at[wid], idx_i)
    pltpu.sync_copy(j_hbm.at[wid], idx_j)

    for k in range(NCHUNKS):
        for t in range(CHUNK // LANES):
            sl = pl.ds(t * LANES, LANES)
            sup_i[k, sl] = _packed_row(idx_i[k, sl])
            sup_j[k, sl] = _packed_row(idx_j[k, sl])

    bufs_i = (bi0, bi1)
    bufs_j = (bj0, bj1)
    sems_i = (si0, si1)
    sems_j = (sj0, sj1)

    def fire(k):
        b = k % 2
        return (pltpu.async_copy(emb_hbm.at[sup_i.at[k]], bufs_i[b], sems_i[b]),
                pltpu.async_copy(emb_hbm.at[sup_j.at[k]], bufs_j[b], sems_j[b]))

    lane = lax.broadcasted_iota(jnp.int32, (LANES,), 0)
    inflight = fire(0)

    for k in range(NCHUNKS):
        b = k % 2
        for cp in inflight:
            cp.wait()
        if k + 1 < NCHUNKS:
            inflight = fire(k + 1)
        bi, bj = bufs_i[b], bufs_j[b]

        def group(g, _):
            sl = pl.ds(g * LANES, LANES)
            hi = (lax.shift_right_logical(idx_i[k, sl], TR_HALF_LOG2) & 1) \
                * EMBED_DIM
            hj = (lax.shift_right_logical(idx_j[k, sl], TR_HALF_LOG2) & 1) \
                * EMBED_DIM
            out_vec = jnp.zeros((LANES,), jnp.float32)
            for t in range(LANES):
                r = g * LANES + t
                hb_i = hi[t]
                hb_j = hj[t]
                acc = jnp.zeros((LANES,), jnp.float32)
                for d in range(EMBED_DIM // LANES):
                    vi = bi[r, pl.ds(hb_i + d * LANES, LANES)]
                    vj = bj[r, pl.ds(hb_j + d * LANES, LANES)]
                    acc = acc + vi * vj
                dot = jnp.sum(acc)
                out_vec = jnp.where(lane == t, dot, out_vec)
            out_v[pl.ds(k * CHUNK + g * LANES, LANES)] = out_vec
            return 0

        lax.fori_loop(0, GROUPS, group, 0)

    pltpu.sync_copy(out_v, out_hbm.at[pl.ds(wid * BPW, BPW)])


@jax.jit
def _sc_dot(i, j, node_emb):
    mesh = plsc.VectorSubcoreMesh(core_axis_name="c", subcore_axis_name="s")
    kfn = pl.kernel(
        _dot_body,
        mesh=mesh,
        compiler_params=pltpu.CompilerParams(
            needs_layout_passes=False, use_tc_tiling_on_sc=True),
        out_type=jax.ShapeDtypeStruct((BATCH,), jnp.float32),
        scratch_types=[
            pltpu.VMEM((NCHUNKS, CHUNK), jnp.int32),
            pltpu.VMEM((NCHUNKS, CHUNK), jnp.int32),
            pltpu.VMEM((NCHUNKS, CHUNK), jnp.int32),
            pltpu.VMEM((NCHUNKS, CHUNK), jnp.int32),
            pltpu.VMEM((CHUNK, PACK_DIM), jnp.float32),
            pltpu.VMEM((CHUNK, PACK_DIM), jnp.float32),
            pltpu.VMEM((CHUNK, PACK_DIM), jnp.float32),
            pltpu.VMEM((CHUNK, PACK_DIM), jnp.float32),
            pltpu.VMEM((BPW,), jnp.float32),
            pltpu.SemaphoreType.DMA,
            pltpu.SemaphoreType.DMA,
            pltpu.SemaphoreType.DMA,
            pltpu.SemaphoreType.DMA,
        ],
    )
    packed = _pack_table(node_emb.T)
    return kfn(i.reshape(NUM_WORKERS, NCHUNKS, CHUNK),
               j.reshape(NUM_WORKERS, NCHUNKS, CHUNK),
               packed)


def kernel(i, j, node_emb):
    return _sc_dot(i.astype(jnp.int32), j.astype(jnp.int32), node_emb)


# speedup vs baseline: 1.7135x; 1.0624x over previous
import functools

import jax
import jax.numpy as jnp
from jax import lax
from jax.experimental import pallas as pl
from jax.experimental.pallas import tpu as pltpu
from jax.experimental.pallas import tpu_sc as plsc

BATCH = 16384
EMBED_DIM = 64
PACK_DIM = 128
LANES = 16
NUM_CORES = 2
NUM_SUBCORES = 16
NUM_WORKERS = NUM_CORES * NUM_SUBCORES
BPW = BATCH // NUM_WORKERS
CHUNK = 128
NCHUNKS = BPW // CHUNK
GROUPS = CHUNK // LANES

TR_BK = 32768
TR_HALF = TR_BK // 2
NUM_NODES = 1000000
TR_GRID = (NUM_NODES + TR_BK - 1) // TR_BK
PACK_ROWS = TR_GRID * TR_HALF


def _tr_body(x_ref, out_ref):
    xt = lax.transpose(x_ref[...], (1, 0))
    out_ref[...] = jnp.concatenate([xt[:TR_HALF], xt[TR_HALF:]], axis=1)


def _pack_table(emb_t):
    return pl.pallas_call(
        _tr_body,
        grid=(TR_GRID,),
        in_specs=[pl.BlockSpec((EMBED_DIM, TR_BK), lambda g: (0, g))],
        out_specs=pl.BlockSpec((TR_HALF, PACK_DIM), lambda g: (g, 0)),
        out_shape=jax.ShapeDtypeStruct((PACK_ROWS, PACK_DIM), jnp.float32),
    )(emb_t)


TR_BK_LOG2 = TR_BK.bit_length() - 1
TR_HALF_LOG2 = TR_HALF.bit_length() - 1


def _packed_row(n):
    return lax.shift_left(
        lax.shift_right_logical(n, TR_BK_LOG2), TR_HALF_LOG2
    ) + (n & (TR_HALF - 1))


def _dot_body(i_hbm, j_hbm, emb_hbm, out_hbm,
              idx_i, idx_j, sup_i, sup_j,
              bi0, bi1, bj0, bj1, out_v,
              si0, si1, sj0, sj1):
    c = lax.axis_index("c")
    s = lax.axis_index("s")
    wid = s * NUM_CORES + c

    pltpu.sync_copy(i_hbm.at[wid], idx_i)
    pltpu.sync_copy(j_hbm.at[wid], idx_j)

    for k in range(NCHUNKS):
        for t in range(CHUNK // LANES):
            sl = pl.ds(t * LANES, LANES)
            sup_i[k, sl] = _packed_row(idx_i[k, sl])
            sup_j[k, sl] = _packed_row(idx_j[k, sl])

    bufs_i = (bi0, bi1)
    bufs_j = (bj0, bj1)
    sems_i = (si0, si1)
    sems_j = (sj0, sj1)

    def fire(k):
        b = k % 2
        return (pltpu.async_copy(emb_hbm.at[sup_i.at[k]], bufs_i[b], sems_i[b]),
                pltpu.async_copy(emb_hbm.at[sup_j.at[k]], bufs_j[b], sems_j[b]))

    lane = lax.broadcasted_iota(jnp.int32, (LANES,), 0)
    inflight = fire(0)

    for k in range(NCHUNKS):
        b = k % 2
        for cp in inflight:
            cp.wait()
        if k + 1 < NCHUNKS:
            inflight = fire(k + 1)
        bi, bj = bufs_i[b], bufs_j[b]

        def group(g, _):
            sl = pl.ds(g * LANES, LANES)
            hi = (lax.shift_right_logical(idx_i[k, sl], TR_HALF_LOG2) & 1) \
                * EMBED_DIM
            hj = (lax.shift_right_logical(idx_j[k, sl], TR_HALF_LOG2) & 1) \
                * EMBED_DIM
            out_vec = jnp.zeros((LANES,), jnp.float32)
            for t in range(LANES):
                r = g * LANES + t
                hb_i = hi[t]
                hb_j = hj[t]
                acc = jnp.zeros((LANES,), jnp.float32)
                for d in range(EMBED_DIM // LANES):
                    vi = bi[r, pl.ds(hb_i + d * LANES, LANES)]
                    vj = bj[r, pl.ds(hb_j + d * LANES, LANES)]
                    acc = acc + vi * vj
                dot = jnp.sum(acc)
                out_vec = jnp.where(lane == t, dot, out_vec)
            out_v[pl.ds(k * CHUNK + g * LANES, LANES)] = out_vec
            return 0

        lax.fori_loop(0, GROUPS, group, 0)

    pltpu.sync_copy(out_v, out_hbm.at[pl.ds(wid * BPW, BPW)])


@jax.jit
def _sc_dot(i, j, node_emb):
    mesh = plsc.VectorSubcoreMesh(core_axis_name="c", subcore_axis_name="s")
    kfn = pl.kernel(
        _dot_body,
        mesh=mesh,
        compiler_params=pltpu.CompilerParams(
            needs_layout_passes=False, use_tc_tiling_on_sc=True),
        out_type=jax.ShapeDtypeStruct((BATCH,), jnp.float32),
        scratch_types=[
            pltpu.VMEM((NCHUNKS, CHUNK), jnp.int32),
            pltpu.VMEM((NCHUNKS, CHUNK), jnp.int32),
            pltpu.VMEM((NCHUNKS, CHUNK), jnp.int32),
            pltpu.VMEM((NCHUNKS, CHUNK), jnp.int32),
            pltpu.VMEM((CHUNK, PACK_DIM), jnp.float32),
            pltpu.VMEM((CHUNK, PACK_DIM), jnp.float32),
            pltpu.VMEM((CHUNK, PACK_DIM), jnp.float32),
            pltpu.VMEM((CHUNK, PACK_DIM), jnp.float32),
            pltpu.VMEM((BPW,), jnp.float32),
            pltpu.SemaphoreType.DMA,
            pltpu.SemaphoreType.DMA,
            pltpu.SemaphoreType.DMA,
            pltpu.SemaphoreType.DMA,
        ],
    )
    packed = _pack_table(node_emb.T)
    return kfn(i.reshape(NUM_WORKERS, NCHUNKS, CHUNK),
               j.reshape(NUM_WORKERS, NCHUNKS, CHUNK),
               packed)


def kernel(i, j, node_emb):
    return _sc_dot(i.astype(jnp.int32), j.astype(jnp.int32), node_emb)
